# baseline (speedup 1.0000x reference)
import functools

import jax
import jax.numpy as jnp
from jax import lax
from jax.experimental import pallas as pl
from jax.experimental.pallas import tpu as pltpu
from jax.experimental.pallas import tpu_sc as plsc

B = 16384
U = 1000000
DG = 16
DM = 64

NC = 2
NS = 16
NW = NC * NS
BPW = B // NW
CHUNK = 128
NCH = BPW // CHUNK
GROUP = 8


def _gather_table(tbl_hbm, idx_v, dat_v, sem, nfeat):
  nstream = nfeat * NCH
  for g0 in range(0, nstream, GROUP):
    for s in range(g0, g0 + GROUP):
      j, c = s // NCH, s % NCH
      sl = pl.ds(c * CHUNK, CHUNK)
      pltpu.async_copy(tbl_hbm.at[idx_v.at[j, sl]], dat_v.at[j, sl], sem)
    for _ in range(GROUP):
      pltpu.make_async_copy(tbl_hbm.at[pl.ds(0, CHUNK)],
                            dat_v.at[0, pl.ds(0, CHUNK)], sem).wait()


def _sc_gather_body(iug_hbm, iig_hbm, ium_hbm, iim_hbm,
                    tug_hbm, tig_hbm, tum_hbm, tim_hbm,
                    ug_out, ig_out, um_out, im_out,
                    idx_v, dat_v, sem):
  wid = lax.axis_index("s") * NC + lax.axis_index("c")
  base = wid * BPW
  col = pl.ds(base, BPW)
  for idx_hbm, tbl_hbm, out_hbm, nfeat in (
      (iug_hbm, tug_hbm, ug_out, DG),
      (iig_hbm, tig_hbm, ig_out, DG),
      (ium_hbm, tum_hbm, um_out, DM),
      (iim_hbm, tim_hbm, im_out, DM),
  ):
    rows = pl.ds(0, nfeat)
    pltpu.sync_copy(idx_hbm.at[rows, col], idx_v.at[rows, :])
    _gather_table(tbl_hbm, idx_v, dat_v, sem, nfeat)
    pltpu.sync_copy(dat_v.at[rows, :], out_hbm.at[rows, col])


_sc_gather = functools.partial(
    pl.kernel,
    out_type=[
        jax.ShapeDtypeStruct((DG, B), jnp.float32),
        jax.ShapeDtypeStruct((DG, B), jnp.float32),
        jax.ShapeDtypeStruct((DM, B), jnp.float32),
        jax.ShapeDtypeStruct((DM, B), jnp.float32),
    ],
    mesh=plsc.VectorSubcoreMesh(core_axis_name="c", subcore_axis_name="s"),
    scratch_types=[
        pltpu.VMEM((DM, BPW), jnp.int32),
        pltpu.VMEM((DM, BPW), jnp.float32),
        pltpu.SemaphoreType.DMA,
    ],
    compiler_params=pltpu.CompilerParams(use_tc_tiling_on_sc=False),
)(_sc_gather_body)


TC_BLK = 1024


def _tc_mlp_body(ug_ref, ig_ref, um_ref, im_ref,
                 w0u_ref, w0i_ref, b0_ref, w1_ref, b1_ref, w2_ref, b2_ref,
                 wfg_ref, wfm_ref, bf_ref, out_ref):
  h = jnp.dot(w0u_ref[...], um_ref[...], preferred_element_type=jnp.float32)
  h += jnp.dot(w0i_ref[...], im_ref[...], preferred_element_type=jnp.float32)
  h = jnp.maximum(h + b0_ref[...], 0.0)
  h = jnp.maximum(
      jnp.dot(w1_ref[...], h, preferred_element_type=jnp.float32) + b1_ref[...], 0.0)
  h = jnp.maximum(
      jnp.dot(w2_ref[...], h, preferred_element_type=jnp.float32) + b2_ref[...], 0.0)
  g = ug_ref[...] * ig_ref[...]
  res = jnp.sum(g * wfg_ref[...], axis=0, keepdims=True)
  res += jnp.sum(h * wfm_ref[...], axis=0, keepdims=True)
  out_ref[0] = res + bf_ref[...]


def _tc_mlp(ug, ig, um, im, W0, b0, W1, b1, W2, b2, Wf, bf):
  nblk = B // TC_BLK
  col_spec = lambda d: pl.BlockSpec((d, TC_BLK), lambda i: (0, i))
  full_spec = lambda s: pl.BlockSpec(s, lambda i: tuple(0 for _ in s))
  return pl.pallas_call(
      _tc_mlp_body,
      grid=(nblk,),
      in_specs=[
          col_spec(DG), col_spec(DG), col_spec(DM), col_spec(DM),
          full_spec((DM, DM)), full_spec((DM, DM)), full_spec((DM, 1)),
          full_spec((DM // 2, DM)), full_spec((DM // 2, 1)),
          full_spec((DM // 4, DM // 2)), full_spec((DM // 4, 1)),
          full_spec((DG, 1)), full_spec((DG, 1)), full_spec((1, 1)),
      ],
      out_specs=pl.BlockSpec((1, 1, TC_BLK), lambda i: (i, 0, 0)),
      out_shape=jax.ShapeDtypeStruct((nblk, 1, TC_BLK), jnp.float32),
  )(ug, ig, um, im,
    W0[:DM].T, W0[DM:].T, b0.reshape(DM, 1),
    W1.T, b1.reshape(DM // 2, 1), W2.T, b2.reshape(DM // 4, 1),
    Wf[:DG], Wf[DG:], bf.reshape(1, 1))


@jax.jit
def kernel(x, eu_gmf, ei_gmf, eu_mlp, ei_mlp, W0, b0, W1, b1, W2, b2, Wf, bf):
  uid = x[:, 0]
  iid = x[:, 1]
  offg = (jnp.arange(DG, dtype=jnp.int32) * U)[:, None]
  offm = (jnp.arange(DM, dtype=jnp.int32) * U)[:, None]
  iug = uid[None, :] + offg
  iig = iid[None, :] + offg
  ium = uid[None, :] + offm
  iim = iid[None, :] + offm
  tug = eu_gmf.T.reshape(-1)
  tig = ei_gmf.T.reshape(-1)
  tum = eu_mlp.T.reshape(-1)
  tim = ei_mlp.T.reshape(-1)
  ug, ig, um, im = _sc_gather(iug, iig, ium, iim, tug, tig, tum, tim)
  out = _tc_mlp(ug, ig, um, im, W0, b0, W1, b1, W2, b2, Wf, bf)
  return out.reshape(-1)


# speedup vs baseline: 6.9926x; 6.9926x over previous
import functools

import jax
import jax.numpy as jnp
from jax import lax
from jax.experimental import pallas as pl
from jax.experimental.pallas import tpu as pltpu
from jax.experimental.pallas import tpu_sc as plsc

B = 16384
DG = 16
DM = 64

NC = 2
NS = 16
NW = NC * NS
BPW = B // NW
CHUNK = 128
NCHUNK = BPW // CHUNK


def _linearize(t, dtype):
  n, d = t.shape
  t = t.astype(dtype)
  t = jax.lax.optimization_barrier(t.reshape(n * d // 128, 128))
  return t.reshape(n, d)


def _sc_gather_body(uid_hbm, iid_hbm, eug_hbm, eig_hbm, eum_hbm, eim_hbm,
                    ug_out, ig_out, um_out, im_out,
                    uidx_v, iidx_v, ug_v, ig_v, um_v, im_v, sem):
  wid = lax.axis_index("s") * NC + lax.axis_index("c")
  base = wid * BPW
  pltpu.sync_copy(uid_hbm.at[pl.ds(wid * NCHUNK, NCHUNK), :], uidx_v)
  pltpu.sync_copy(iid_hbm.at[pl.ds(wid * NCHUNK, NCHUNK), :], iidx_v)
  copies = []
  for j in range(NCHUNK):
    sl = pl.ds(j * CHUNK, CHUNK)
    copies.append(pltpu.async_copy(eug_hbm.at[uidx_v.at[j]], ug_v.at[sl, :], sem))
    copies.append(pltpu.async_copy(eig_hbm.at[iidx_v.at[j]], ig_v.at[sl, :], sem))
    copies.append(pltpu.async_copy(eum_hbm.at[uidx_v.at[j]], um_v.at[sl, :], sem))
    copies.append(pltpu.async_copy(eim_hbm.at[iidx_v.at[j]], im_v.at[sl, :], sem))
  for c in copies:
    c.wait()
  pltpu.sync_copy(ug_v, ug_out.at[pl.ds(base, BPW), :])
  pltpu.sync_copy(ig_v, ig_out.at[pl.ds(base, BPW), :])
  pltpu.sync_copy(um_v, um_out.at[pl.ds(base, BPW), :])
  pltpu.sync_copy(im_v, im_out.at[pl.ds(base, BPW), :])


_sc_gather = functools.partial(
    pl.kernel,
    out_type=[
        jax.ShapeDtypeStruct((B, DG), jnp.float32),
        jax.ShapeDtypeStruct((B, DG), jnp.float32),
        jax.ShapeDtypeStruct((B, DM), jnp.float32),
        jax.ShapeDtypeStruct((B, DM), jnp.float32),
    ],
    mesh=plsc.VectorSubcoreMesh(core_axis_name="c", subcore_axis_name="s"),
    scratch_types=[
        pltpu.VMEM((NCHUNK, CHUNK), jnp.int32),
        pltpu.VMEM((NCHUNK, CHUNK), jnp.int32),
        pltpu.VMEM((BPW, DG), jnp.float32),
        pltpu.VMEM((BPW, DG), jnp.float32),
        pltpu.VMEM((BPW, DM), jnp.float32),
        pltpu.VMEM((BPW, DM), jnp.float32),
        pltpu.SemaphoreType.DMA,
    ],
    compiler_params=pltpu.CompilerParams(use_tc_tiling_on_sc=False),
)(_sc_gather_body)


TC_BLK = 1024


def _tc_mlp_body(ug_ref, ig_ref, um_ref, im_ref,
                 w0_ref, b0_ref, w1_ref, b1_ref, w2_ref, b2_ref,
                 wf_ref, bf_ref, out_ref):
  w0 = w0_ref[...]
  um = um_ref[...].astype(jnp.float32)
  im = im_ref[...].astype(jnp.float32)
  h = jnp.dot(um, w0[:DM], preferred_element_type=jnp.float32)
  h += jnp.dot(im, w0[DM:], preferred_element_type=jnp.float32)
  h = jnp.maximum(h + b0_ref[...], 0.0)
  h = jnp.maximum(
      jnp.dot(h, w1_ref[...], preferred_element_type=jnp.float32) + b1_ref[...], 0.0)
  h = jnp.maximum(
      jnp.dot(h, w2_ref[...], preferred_element_type=jnp.float32) + b2_ref[...], 0.0)
  g = ug_ref[...] * ig_ref[...]
  wf = wf_ref[...]
  res = jnp.dot(g, wf[:DG], preferred_element_type=jnp.float32)
  res += jnp.dot(h, wf[DG:], preferred_element_type=jnp.float32)
  out_ref[...] = res + bf_ref[...]


def _tc_mlp(ug, ig, um, im, W0, b0, W1, b1, W2, b2, Wf, bf):
  nblk = B // TC_BLK
  row_spec = lambda d: pl.BlockSpec((TC_BLK, d), lambda i: (i, 0))
  full_spec = lambda s: pl.BlockSpec(s, lambda i: tuple(0 for _ in s))
  return pl.pallas_call(
      _tc_mlp_body,
      grid=(nblk,),
      in_specs=[
          row_spec(DG), row_spec(DG), row_spec(DM), row_spec(DM),
          full_spec((2 * DM, DM)), full_spec((1, DM)),
          full_spec((DM, DM // 2)), full_spec((1, DM // 2)),
          full_spec((DM // 2, DM // 4)), full_spec((1, DM // 4)),
          full_spec((2 * DG, 1)), full_spec((1, 1)),
      ],
      out_specs=pl.BlockSpec((TC_BLK, 1), lambda i: (i, 0)),
      out_shape=jax.ShapeDtypeStruct((B, 1), jnp.float32),
  )(ug, ig, um, im, W0, b0.reshape(1, DM), W1, b1.reshape(1, DM // 2),
    W2, b2.reshape(1, DM // 4), Wf, bf.reshape(1, 1))


@jax.jit
def kernel(x, eu_gmf, ei_gmf, eu_mlp, ei_mlp, W0, b0, W1, b1, W2, b2, Wf, bf):
  uid = x[:, 0].reshape(B // CHUNK, CHUNK)
  iid = x[:, 1].reshape(B // CHUNK, CHUNK)
  eu_gmf = _linearize(eu_gmf, jnp.float32)
  ei_gmf = _linearize(ei_gmf, jnp.float32)
  eu_mlp = _linearize(eu_mlp, jnp.float32)
  ei_mlp = _linearize(ei_mlp, jnp.float32)
  ug, ig, um, im = _sc_gather(uid, iid, eu_gmf, ei_gmf, eu_mlp, ei_mlp)
  out = _tc_mlp(ug, ig, um, im, W0, b0, W1, b1, W2, b2, Wf, bf)
  return out.reshape(-1)
